# baseline (speedup 1.0000x reference)
import functools

import jax
import jax.numpy as jnp
from jax import lax
from jax.experimental import pallas as pl
from jax.experimental.pallas import tpu as pltpu
from jax.experimental.pallas import tpu_sc as plsc

N = 10000
E = 320000
D = 128

NC = 2
NS = 16
NW = NC * NS
CHUNK = 128
CPT = 80
E_PAD = NW * CPT * CHUNK

N_ACC = 10240
ZROWS = N_ACC // NS
WROWS = 632
N_OUT = NS * WROWS


def _sc_body(x_hbm, col_hbm, row_hbm, w_hbm, out_hbm,
             wv, rows0, rows1, cb0, cb1, rb0, rb1, accum,
             gsem0, gsem1, ssem0, ssem1, csem0, csem1, rsem0, rsem1):
    c = lax.axis_index("c")
    s = lax.axis_index("s")
    wid = s * NC + c
    ebase0 = wid * (CPT * CHUNK)

    with jax.named_scope("stage"):
        pltpu.sync_copy(w_hbm.at[wid], wv)

    zero16 = jnp.zeros((16,), jnp.float32)

    def zrow(i, carry):
        for t in range(D // 16):
            rows0[i, pl.ds(t * 16, 16)] = zero16
        return carry

    with jax.named_scope("zero"):
        lax.fori_loop(0, CHUNK, zrow, 0)
        for rep in range(ZROWS // CHUNK):
            pltpu.sync_copy(
                rows0, accum.at[pl.ds(s * ZROWS + rep * CHUNK, CHUNK)])
    with jax.named_scope("bar1"):
        plsc.subcore_barrier()

    bufs = ((rows0, cb0, rb0, gsem0, ssem0, csem0, rsem0),
            (rows1, cb1, rb1, gsem1, ssem1, csem1, rsem1))

    def col_dma(j, dst, sem):
        return pltpu.make_async_copy(
            col_hbm.at[pl.ds(ebase0 + j * CHUNK, CHUNK)], dst, sem)

    def row_dma(j, dst, sem):
        return pltpu.make_async_copy(
            row_hbm.at[pl.ds(ebase0 + j * CHUNK, CHUNK)], dst, sem)

    def scale(rbuf, j):
        jbase = j * CHUNK

        def group_body(g, gcarry):
            ebase = g * 16
            wvec = wv[pl.ds(jbase + ebase, 16)]
            for l in range(16):
                w16 = jnp.broadcast_to(wvec[l], (16,))
                e = ebase + l
                for t in range(D // 16):
                    sl = pl.ds(t * 16, 16)
                    rbuf[e, sl] = rbuf[e, sl] * w16
            return gcarry

        lax.fori_loop(0, CHUNK // 16, group_body, 0)

    col_dma(0, cb0, csem0).start()
    col_dma(1, cb1, csem1).start()
    row_dma(0, rb0, rsem0).start()
    col_dma(0, cb0, csem0).wait()
    pltpu.async_copy(x_hbm.at[cb0], rows0, gsem0)

    def pair_body(jj, carry):
        for b in range(2):
            j = jj * 2 + b
            rbuf, cbuf, rbI, gsem, ssem, csem, rsem = bufs[b]
            obuf, ocbuf, orbI, ogsem, ossem, ocsem, orsem = bufs[1 - b]

            @pl.when(j + 1 < CPT)
            def _():
                row_dma(j + 1, orbI, orsem).start()
                col_dma(j + 1, ocbuf, ocsem).wait()
                pltpu.async_copy(x_hbm.at[ocbuf], obuf, ogsem)

            with jax.named_scope("gwait"):
                pltpu.make_async_copy(x_hbm.at[cbuf], rbuf, gsem).wait()

            @pl.when(j + 2 < CPT)
            def _():
                col_dma(j + 2, cbuf, csem).start()

            with jax.named_scope("scale"):
                scale(rbuf, j)
            with jax.named_scope("scat"):
                row_dma(j, rbI, rsem).wait()
                pltpu.async_copy(rbuf, accum.at[rbI], ssem, add=True)
                pltpu.make_async_copy(rbuf, accum.at[rbI], ssem).wait()
        return carry

    lax.fori_loop(0, CPT // 2, pair_body, 0)
    with jax.named_scope("bar2"):
        plsc.subcore_barrier()

    with jax.named_scope("wout"):
        pltpu.sync_copy(accum.at[pl.ds(s * WROWS, WROWS)],
                        out_hbm.at[c, pl.ds(s * WROWS, WROWS)])


@jax.jit
def _sc_spmm(x, col3, row3, w3):
    mesh = plsc.VectorSubcoreMesh(core_axis_name="c", subcore_axis_name="s")
    return pl.kernel(
        _sc_body,
        mesh=mesh,
        out_type=jax.ShapeDtypeStruct((NC, N_OUT, D), jnp.float32),
        scratch_types=[
            pltpu.VMEM((CPT * CHUNK,), jnp.float32),
            pltpu.VMEM((CHUNK, D), jnp.float32),
            pltpu.VMEM((CHUNK, D), jnp.float32),
            pltpu.VMEM((CHUNK,), jnp.int32),
            pltpu.VMEM((CHUNK,), jnp.int32),
            pltpu.VMEM((CHUNK,), jnp.int32),
            pltpu.VMEM((CHUNK,), jnp.int32),
            pltpu.VMEM_SHARED((N_ACC, D), jnp.float32),
        ] + [pltpu.SemaphoreType.DMA] * 8,
    )(x, col3, row3, w3)


def _fin_body(a_ref, c1_ref, b_ref, p_ref, x_ref, o_ref):
    ax = p_ref[0] + p_ref[1]
    xv = x_ref[...]
    o_ref[...] = a_ref[0, 0] * ax + c1_ref[0, 0] * xv - b_ref[0, 0] * (xv * xv)


BR = 2000


@jax.jit
def _finalize(p, x, alpha, c1, beta):
    sspec = pl.BlockSpec(memory_space=pltpu.SMEM)
    return pl.pallas_call(
        _fin_body,
        grid=(N // BR,),
        in_specs=[
            sspec,
            sspec,
            sspec,
            pl.BlockSpec((NC, BR, D), lambda i: (0, i, 0)),
            pl.BlockSpec((BR, D), lambda i: (i, 0)),
        ],
        out_specs=pl.BlockSpec((BR, D), lambda i: (i, 0)),
        out_shape=jax.ShapeDtypeStruct((N, D), jnp.float32),
    )(alpha.reshape(1, 1), c1.reshape(1, 1), beta.reshape(1, 1), p, x)


def kernel(t, x, edge_index, edge_weight, alpha_train, beta_train, source_train):
    row = edge_index[0]
    col = edge_index[1]
    pad = E_PAD - E
    col3 = jnp.concatenate([col, jnp.zeros((pad,), jnp.int32)])
    row3 = jnp.concatenate([row, jnp.zeros((pad,), jnp.int32)])
    w3 = jnp.concatenate(
        [edge_weight, jnp.zeros((pad,), jnp.float32)]).reshape(NW, CPT * CHUNK)

    partials = _sc_spmm(x, col3, row3, w3)

    alpha = jax.nn.sigmoid(alpha_train) * 0.1
    beta = jax.nn.sigmoid(beta_train) * 0.1
    c1 = beta - alpha + 0.1 * source_train
    return _finalize(partials, x, alpha.astype(jnp.float32),
                     c1.astype(jnp.float32), beta.astype(jnp.float32))


# speedup vs baseline: 3.1145x; 3.1145x over previous
import functools

import jax
import jax.numpy as jnp
from jax import lax
from jax.experimental import pallas as pl
from jax.experimental.pallas import tpu as pltpu
from jax.experimental.pallas import tpu_sc as plsc

N = 10000
E = 320000
D = 128

NC = 2
NS = 16
NW = NC * NS
CHUNK = 128
CPT = 80
E_PAD = NW * CPT * CHUNK

N_ACC = 10240
ZROWS = N_ACC // NS
WROWS = 632
N_OUT = NS * WROWS


def _sc_body(x_hbm, col_hbm, row_hbm, w_hbm, out_hbm,
             wv, rows0, rows1, cb0, cb1, rb0, rb1, accum,
             gsem0, gsem1, ssem0, ssem1, csem0, csem1, rsem0, rsem1):
    c = lax.axis_index("c")
    s = lax.axis_index("s")
    wid = s * NC + c
    ebase0 = wid * (CPT * CHUNK)

    with jax.named_scope("stage"):
        pltpu.sync_copy(w_hbm.at[wid], wv)

    zero16 = jnp.zeros((16,), jnp.float32)

    def zrow(i, carry):
        for t in range(D // 16):
            rows0[i, pl.ds(t * 16, 16)] = zero16
        return carry

    with jax.named_scope("zero"):
        lax.fori_loop(0, CHUNK, zrow, 0)
        for rep in range(ZROWS // CHUNK):
            pltpu.sync_copy(
                rows0, accum.at[pl.ds(s * ZROWS + rep * CHUNK, CHUNK)])
    with jax.named_scope("bar1"):
        plsc.subcore_barrier()

    bufs = ((rows0, cb0, rb0, gsem0, ssem0, csem0, rsem0),
            (rows1, cb1, rb1, gsem1, ssem1, csem1, rsem1))

    def col_dma(j, dst, sem):
        return pltpu.make_async_copy(
            col_hbm.at[pl.ds(ebase0 + j * CHUNK, CHUNK)], dst, sem)

    def row_dma(j, dst, sem):
        return pltpu.make_async_copy(
            row_hbm.at[pl.ds(ebase0 + j * CHUNK, CHUNK)], dst, sem)

    def scale(rbuf, j):
        jbase = j * CHUNK

        def group_body(g, gcarry):
            ebase = g * 16
            wvec = wv[pl.ds(jbase + ebase, 16)]
            for l in range(16):
                w16 = jnp.broadcast_to(wvec[l], (16,))
                e = ebase + l
                for t in range(D // 16):
                    sl = pl.ds(t * 16, 16)
                    rbuf[e, sl] = rbuf[e, sl] * w16
            return gcarry

        lax.fori_loop(0, CHUNK // 16, group_body, 0)

    col_dma(0, cb0, csem0).start()
    col_dma(1, cb1, csem1).start()
    row_dma(0, rb0, rsem0).start()
    col_dma(0, cb0, csem0).wait()
    pltpu.async_copy(x_hbm.at[cb0], rows0, gsem0)

    def pair_body(jj, carry):
        for b in range(2):
            j = jj * 2 + b
            rbuf, cbuf, rbI, gsem, ssem, csem, rsem = bufs[b]
            obuf, ocbuf, orbI, ogsem, ossem, ocsem, orsem = bufs[1 - b]

            @pl.when(j >= 1)
            def _():
                pltpu.make_async_copy(
                    obuf, accum.at[orbI], ossem).wait()

            @pl.when(j + 1 < CPT)
            def _():
                row_dma(j + 1, orbI, orsem).start()
                col_dma(j + 1, ocbuf, ocsem).wait()
                pltpu.async_copy(x_hbm.at[ocbuf], obuf, ogsem)

            with jax.named_scope("gwait"):
                pltpu.make_async_copy(x_hbm.at[cbuf], rbuf, gsem).wait()

            @pl.when(j + 2 < CPT)
            def _():
                col_dma(j + 2, cbuf, csem).start()

            with jax.named_scope("scale"):
                scale(rbuf, j)
            with jax.named_scope("scat"):
                row_dma(j, rbI, rsem).wait()
                pltpu.async_copy(rbuf, accum.at[rbI], ssem, add=True)
        return carry

    lax.fori_loop(0, CPT // 2, pair_body, 0)
    pltpu.make_async_copy(bufs[1][0], accum.at[bufs[1][2]], bufs[1][4]).wait()
    with jax.named_scope("bar2"):
        plsc.subcore_barrier()

    with jax.named_scope("wout"):
        pltpu.sync_copy(accum.at[pl.ds(s * WROWS, WROWS)],
                        out_hbm.at[c, pl.ds(s * WROWS, WROWS)])


@jax.jit
def _sc_spmm(x, col3, row3, w3):
    mesh = plsc.VectorSubcoreMesh(core_axis_name="c", subcore_axis_name="s")
    return pl.kernel(
        _sc_body,
        mesh=mesh,
        out_type=jax.ShapeDtypeStruct((NC, N_OUT, D), jnp.float32),
        scratch_types=[
            pltpu.VMEM((CPT * CHUNK,), jnp.float32),
            pltpu.VMEM((CHUNK, D), jnp.float32),
            pltpu.VMEM((CHUNK, D), jnp.float32),
            pltpu.VMEM((CHUNK,), jnp.int32),
            pltpu.VMEM((CHUNK,), jnp.int32),
            pltpu.VMEM((CHUNK,), jnp.int32),
            pltpu.VMEM((CHUNK,), jnp.int32),
            pltpu.VMEM_SHARED((N_ACC, D), jnp.float32),
        ] + [pltpu.SemaphoreType.DMA] * 8,
    )(x, col3, row3, w3)


def _fin_body(a_ref, c1_ref, b_ref, p_ref, x_ref, o_ref):
    ax = p_ref[0] + p_ref[1]
    xv = x_ref[...]
    o_ref[...] = a_ref[0, 0] * ax + c1_ref[0, 0] * xv - b_ref[0, 0] * (xv * xv)


BR = 2000


@jax.jit
def _finalize(p, x, alpha, c1, beta):
    sspec = pl.BlockSpec(memory_space=pltpu.SMEM)
    return pl.pallas_call(
        _fin_body,
        grid=(N // BR,),
        in_specs=[
            sspec,
            sspec,
            sspec,
            pl.BlockSpec((NC, BR, D), lambda i: (0, i, 0)),
            pl.BlockSpec((BR, D), lambda i: (i, 0)),
        ],
        out_specs=pl.BlockSpec((BR, D), lambda i: (i, 0)),
        out_shape=jax.ShapeDtypeStruct((N, D), jnp.float32),
    )(alpha.reshape(1, 1), c1.reshape(1, 1), beta.reshape(1, 1), p, x)


def kernel(t, x, edge_index, edge_weight, alpha_train, beta_train, source_train):
    row = edge_index[0]
    col = edge_index[1]
    pad = E_PAD - E
    spread = (jnp.arange(pad, dtype=jnp.int32) * 13) % N
    col3 = jnp.concatenate([col, spread])
    row3 = jnp.concatenate([row, spread])
    w3 = jnp.concatenate(
        [edge_weight, jnp.zeros((pad,), jnp.float32)]).reshape(NW, CPT * CHUNK)

    partials = _sc_spmm(x, col3, row3, w3)

    alpha = jax.nn.sigmoid(alpha_train) * 0.1
    beta = jax.nn.sigmoid(beta_train) * 0.1
    c1 = beta - alpha + 0.1 * source_train
    return _finalize(partials, x, alpha.astype(jnp.float32),
                     c1.astype(jnp.float32), beta.astype(jnp.float32))


# speedup vs baseline: 3.3032x; 1.0606x over previous
import functools

import jax
import jax.numpy as jnp
from jax import lax
from jax.experimental import pallas as pl
from jax.experimental.pallas import tpu as pltpu
from jax.experimental.pallas import tpu_sc as plsc

N = 10000
E = 320000
D = 128

NC = 2
NS = 16
NW = NC * NS
CHUNK = 128
CPT = 81
E_PAD = NW * CPT * CHUNK

N_ACC = N
WROWS = 632
WLAST = N - 15 * WROWS


def _sc_body(x_hbm, col_hbm, row_hbm, w_hbm, out_hbm,
             r0, r1, r2, cb0, cb1, cb2, rb0, rb1, rb2, wb0, wb1, wb2, accum,
             g0, g1, g2, s0, s1, s2, c0, c1, c2, q0, q1, q2, w0, w1, w2):
    c = lax.axis_index("c")
    s = lax.axis_index("s")
    wid = s * NC + c
    ebase0 = wid * (CPT * CHUNK)

    rows = (r0, r1, r2)
    cb = (cb0, cb1, cb2)
    rb = (rb0, rb1, rb2)
    wb = (wb0, wb1, wb2)
    gsem = (g0, g1, g2)
    ssem = (s0, s1, s2)
    csem = (c0, c1, c2)
    rsem = (q0, q1, q2)
    wsem = (w0, w1, w2)

    zero16 = jnp.zeros((16,), jnp.float32)

    def zrow(i, carry):
        for t in range(D // 16):
            r0[i, pl.ds(t * 16, 16)] = zero16
        return carry

    with jax.named_scope("zero"):
        lax.fori_loop(0, CHUNK, zrow, 0)
        zbase = s * 625
        for rep in range(4):
            pltpu.sync_copy(r0, accum.at[pl.ds(zbase + rep * CHUNK, CHUNK)])
        pltpu.sync_copy(r0.at[pl.ds(0, 113)],
                        accum.at[pl.ds(zbase + 512, 113)])
    with jax.named_scope("bar1"):
        plsc.subcore_barrier()

    def col_dma(j, k):
        return pltpu.make_async_copy(
            col_hbm.at[pl.ds(ebase0 + j * CHUNK, CHUNK)], cb[k], csem[k])

    def row_dma(j, k):
        return pltpu.make_async_copy(
            row_hbm.at[pl.ds(ebase0 + j * CHUNK, CHUNK)], rb[k], rsem[k])

    def w_dma(j, k):
        return pltpu.make_async_copy(
            w_hbm.at[pl.ds(ebase0 + j * CHUNK, CHUNK)], wb[k], wsem[k])

    def gat_dma(k):
        return pltpu.make_async_copy(x_hbm.at[cb[k]], rows[k], gsem[k])

    def scat_start(k):
        pltpu.async_copy(rows[k], accum.at[rb[k]], ssem[k], add=True)

    def scat_wait(k):
        pltpu.make_async_copy(rows[k], accum.at[rb[k]], ssem[k]).wait()

    def scale(k):
        rbuf, wbuf = rows[k], wb[k]

        def group_body(g, gcarry):
            ebase = g * 16
            wvec = wbuf[pl.ds(ebase, 16)]
            for l in range(16):
                w16 = jnp.broadcast_to(wvec[l], (16,))
                e = ebase + l
                for t in range(D // 16):
                    sl = pl.ds(t * 16, 16)
                    rbuf[e, sl] = rbuf[e, sl] * w16
            return gcarry

        lax.fori_loop(0, CHUNK // 16, group_body, 0)

    col_dma(0, 0).start()
    col_dma(1, 1).start()
    row_dma(0, 0).start()
    w_dma(0, 0).start()
    w_dma(1, 1).start()
    col_dma(0, 0).wait()
    gat_dma(0).start()

    def tri_body(jg, carry):
        for b in range(3):
            j = jg * 3 + b
            k = b
            kn = (b + 1) % 3
            kp = (b + 2) % 3

            @pl.when(j >= 2)
            def _():
                with jax.named_scope("drain"):
                    scat_wait(kn)

            @pl.when(j + 1 < CPT)
            def _():
                row_dma(j + 1, kn).start()
                col_dma(j + 1, kn).wait()
                gat_dma(kn).start()

            with jax.named_scope("gwait"):
                gat_dma(k).wait()

            @pl.when(j + 2 < CPT)
            def _():
                col_dma(j + 2, kp).start()
                w_dma(j + 2, kp).start()

            with jax.named_scope("scale"):
                w_dma(j, k).wait()
                scale(k)
            with jax.named_scope("scat"):
                row_dma(j, k).wait()
                scat_start(k)
        return carry

    lax.fori_loop(0, CPT // 3, tri_body, 0)
    scat_wait((CPT - 2) % 3)
    scat_wait((CPT - 1) % 3)
    with jax.named_scope("bar2"):
        plsc.subcore_barrier()

    with jax.named_scope("wout"):
        @pl.when(s < 15)
        def _():
            pltpu.sync_copy(accum.at[pl.ds(s * WROWS, WROWS)],
                            out_hbm.at[c, pl.ds(s * WROWS, WROWS)])

        @pl.when(s == 15)
        def _():
            pltpu.sync_copy(accum.at[pl.ds(15 * WROWS, WLAST)],
                            out_hbm.at[c, pl.ds(15 * WROWS, WLAST)])


@jax.jit
def _sc_spmm(x, col3, row3, w3):
    mesh = plsc.VectorSubcoreMesh(core_axis_name="c", subcore_axis_name="s")
    return pl.kernel(
        _sc_body,
        mesh=mesh,
        out_type=jax.ShapeDtypeStruct((NC, N, D), jnp.float32),
        scratch_types=[
            pltpu.VMEM((CHUNK, D), jnp.float32),
            pltpu.VMEM((CHUNK, D), jnp.float32),
            pltpu.VMEM((CHUNK, D), jnp.float32),
            pltpu.VMEM((CHUNK,), jnp.int32),
            pltpu.VMEM((CHUNK,), jnp.int32),
            pltpu.VMEM((CHUNK,), jnp.int32),
            pltpu.VMEM((CHUNK,), jnp.int32),
            pltpu.VMEM((CHUNK,), jnp.int32),
            pltpu.VMEM((CHUNK,), jnp.int32),
            pltpu.VMEM((CHUNK,), jnp.float32),
            pltpu.VMEM((CHUNK,), jnp.float32),
            pltpu.VMEM((CHUNK,), jnp.float32),
            pltpu.VMEM_SHARED((N_ACC, D), jnp.float32),
        ] + [pltpu.SemaphoreType.DMA] * 15,
    )(x, col3, row3, w3)


def _fin_body(a_ref, c1_ref, b_ref, p_ref, x_ref, o_ref):
    ax = p_ref[0] + p_ref[1]
    xv = x_ref[...]
    o_ref[...] = a_ref[0, 0] * ax + c1_ref[0, 0] * xv - b_ref[0, 0] * (xv * xv)


BR = 2000


@jax.jit
def _finalize(p, x, alpha, c1, beta):
    sspec = pl.BlockSpec(memory_space=pltpu.SMEM)
    return pl.pallas_call(
        _fin_body,
        grid=(N // BR,),
        in_specs=[
            sspec,
            sspec,
            sspec,
            pl.BlockSpec((NC, BR, D), lambda i: (0, i, 0)),
            pl.BlockSpec((BR, D), lambda i: (i, 0)),
        ],
        out_specs=pl.BlockSpec((BR, D), lambda i: (i, 0)),
        out_shape=jax.ShapeDtypeStruct((N, D), jnp.float32),
    )(alpha.reshape(1, 1), c1.reshape(1, 1), beta.reshape(1, 1), p, x)


def kernel(t, x, edge_index, edge_weight, alpha_train, beta_train, source_train):
    row = edge_index[0]
    col = edge_index[1]
    pad = E_PAD - E
    spread = (jnp.arange(pad, dtype=jnp.int32) * 13) % N
    col3 = jnp.concatenate([col, spread])
    row3 = jnp.concatenate([row, spread])
    w3 = jnp.concatenate([edge_weight, jnp.zeros((pad,), jnp.float32)])

    partials = _sc_spmm(x, col3, row3, w3)

    alpha = jax.nn.sigmoid(alpha_train) * 0.1
    beta = jax.nn.sigmoid(beta_train) * 0.1
    c1 = beta - alpha + 0.1 * source_train
    return _finalize(partials, x, alpha.astype(jnp.float32),
                     c1.astype(jnp.float32), beta.astype(jnp.float32))
